# baseline (speedup 1.0000x reference)
import jax
import jax.numpy as jnp
from jax.experimental import pallas as pl


def _pos_body(col_ref, row_ref, out_ref):
    colT = col_ref[:].T
    rowT = row_ref[:].T
    out_ref[0, 0:256] = jnp.broadcast_to(colT[:, None, :], (256, 64, 64))
    out_ref[0, 256:512] = jnp.broadcast_to(rowT[:, :, None], (256, 64, 64))


def kernel(x, row_embed, col_embed):
    b, _, h, w = x.shape
    f = col_embed.shape[-1]
    return pl.pallas_call(
        _pos_body,
        grid=(b,),
        in_specs=[
            pl.BlockSpec((w, f), lambda i: (0, 0)),
            pl.BlockSpec((h, f), lambda i: (0, 0)),
        ],
        out_specs=pl.BlockSpec((1, 2 * f, h, w), lambda i: (i, 0, 0, 0)),
        out_shape=jax.ShapeDtypeStruct((b, 2 * f, h, w), x.dtype),
    )(col_embed, row_embed)


# speedup vs baseline: 1.6070x; 1.6070x over previous
import jax
import jax.numpy as jnp
from jax.experimental import pallas as pl
from jax.experimental.pallas import tpu as pltpu


def _pos_body(col_ref, row_ref, out_ref, scratch, sem):
    b = out_ref.shape[0]
    f = col_ref.shape[1]
    w = out_ref.shape[2] // col_ref.shape[0]
    h = row_ref.shape[0]

    colT = col_ref[:].T
    scratch[0:f] = jnp.broadcast_to(colT[:, None, :], (f, h, w)).reshape(f, h * w)
    col_copies = [
        pltpu.make_async_copy(scratch.at[0:f], out_ref.at[i, 0:f], sem)
        for i in range(b)
    ]
    for c in col_copies:
        c.start()

    rowT = row_ref[:].T
    scratch[f : 2 * f] = jnp.broadcast_to(rowT[:, :, None], (f, h, w)).reshape(
        f, h * w
    )
    row_copies = [
        pltpu.make_async_copy(scratch.at[f : 2 * f], out_ref.at[i, f : 2 * f], sem)
        for i in range(b)
    ]
    for c in row_copies:
        c.start()

    for c in col_copies:
        c.wait()
    for c in row_copies:
        c.wait()


def kernel(x, row_embed, col_embed):
    b, _, h, w = x.shape
    f = col_embed.shape[-1]
    out_flat = pl.pallas_call(
        _pos_body,
        in_specs=[
            pl.BlockSpec((w, f), lambda: (0, 0)),
            pl.BlockSpec((h, f), lambda: (0, 0)),
        ],
        out_specs=pl.BlockSpec(memory_space=pl.ANY),
        out_shape=jax.ShapeDtypeStruct((b, 2 * f, h * w), x.dtype),
        scratch_shapes=[
            pltpu.VMEM((2 * f, h * w), x.dtype),
            pltpu.SemaphoreType.DMA,
        ],
    )(col_embed[:w], row_embed[:h])
    return out_flat.reshape(b, 2 * f, h, w)


# speedup vs baseline: 1.6114x; 1.0027x over previous
import jax
import jax.numpy as jnp
from jax.experimental import pallas as pl
from jax.experimental.pallas import tpu as pltpu


def _pos_body(col_ref, row_ref, out_ref, scratch, sem):
    b = out_ref.shape[0]
    f = col_ref.shape[1]
    w = out_ref.shape[2] // col_ref.shape[0]
    h = row_ref.shape[0]

    colT = col_ref[:].T
    scratch[0:f] = jnp.broadcast_to(colT[:, None, :], (f, h, w)).reshape(f, h * w)
    chunk = 64
    col_copies = [
        pltpu.make_async_copy(
            scratch.at[c0 : c0 + chunk], out_ref.at[i, c0 : c0 + chunk], sem
        )
        for i in range(b)
        for c0 in range(0, f, chunk)
    ]
    for c in col_copies:
        c.start()

    rowT = row_ref[:].T
    scratch[f : 2 * f] = jnp.broadcast_to(rowT[:, :, None], (f, h, w)).reshape(
        f, h * w
    )
    row_copies = [
        pltpu.make_async_copy(
            scratch.at[f + c0 : f + c0 + chunk],
            out_ref.at[i, f + c0 : f + c0 + chunk],
            sem,
        )
        for i in range(b)
        for c0 in range(0, f, chunk)
    ]
    for c in row_copies:
        c.start()

    for c in col_copies:
        c.wait()
    for c in row_copies:
        c.wait()


def kernel(x, row_embed, col_embed):
    b, _, h, w = x.shape
    f = col_embed.shape[-1]
    out_flat = pl.pallas_call(
        _pos_body,
        in_specs=[
            pl.BlockSpec((w, f), lambda: (0, 0)),
            pl.BlockSpec((h, f), lambda: (0, 0)),
        ],
        out_specs=pl.BlockSpec(memory_space=pl.ANY),
        out_shape=jax.ShapeDtypeStruct((b, 2 * f, h * w), x.dtype),
        scratch_shapes=[
            pltpu.VMEM((2 * f, h * w), x.dtype),
            pltpu.SemaphoreType.DMA,
        ],
    )(col_embed[:w], row_embed[:h])
    return out_flat.reshape(b, 2 * f, h, w)
